# baseline (speedup 1.0000x reference)
import functools

import jax
import jax.numpy as jnp
from jax import lax
from jax.experimental import pallas as pl
from jax.experimental.pallas import tpu as pltpu
from jax.experimental.pallas import tpu_sc as plsc

_VOCAB = 100000
_D = 128
_H = 512
_B = 16384

_NCHUNK = 4
_CB = _B // _NCHUNK

_NC, _NS = 2, 16
_NW = _NC * _NS
_BPW = _CB // _NW
_CROWS = min(_BPW, 128)
_CH = _BPW // _CROWS


def _sc_gather(qtab_hbm, etab_hbm, qids_hbm, eids_hbm, q_out, e_out,
               qidx_v, eidx_v, rows0, rows1, sem0, sem1):
    wid = lax.axis_index("s") * _NC + lax.axis_index("c")
    base = wid * _BPW
    pltpu.sync_copy(qids_hbm.at[pl.ds(base, _BPW)], qidx_v)
    pltpu.sync_copy(eids_hbm.at[pl.ds(base, _BPW)], eidx_v)
    bufs = (rows0, rows1)
    sems = (sem0, sem1)
    work = [(qtab_hbm, qidx_v, q_out, c) for c in range(_CH)]
    work += [(etab_hbm, eidx_v, e_out, c) for c in range(_CH)]

    def _start(j, buf, sem):
        tab, idx, _, c = work[j]
        pltpu.async_copy(tab.at[idx.at[pl.ds(c * _CROWS, _CROWS)]], buf, sem)

    _start(0, bufs[0], sems[0])
    for j in range(len(work)):
        if j + 1 < len(work):
            _start(j + 1, bufs[(j + 1) % 2], sems[(j + 1) % 2])
        tab, idx, out, c = work[j]
        buf = bufs[j % 2]
        pltpu.make_async_copy(tab.at[idx.at[pl.ds(c * _CROWS, _CROWS)]],
                              buf, sems[j % 2]).wait()
        pltpu.sync_copy(buf, out.at[pl.ds(base + c * _CROWS, _CROWS)])


@functools.lru_cache(maxsize=None)
def _gather_call():
    return pl.kernel(
        _sc_gather,
        mesh=plsc.VectorSubcoreMesh(core_axis_name="c", subcore_axis_name="s"),
        out_type=[
            jax.ShapeDtypeStruct((_CB, _D), jnp.float32),
            jax.ShapeDtypeStruct((_CB, _D), jnp.float32),
        ],
        scratch_types=[
            pltpu.VMEM((_BPW,), jnp.int32),
            pltpu.VMEM((_BPW,), jnp.int32),
            pltpu.VMEM((_CROWS, _D), jnp.float32),
            pltpu.VMEM((_CROWS, _D), jnp.float32),
            pltpu.SemaphoreType.DMA,
            pltpu.SemaphoreType.DMA,
        ],
    )


_BLK = 2048
_GSTEPS = _CB // _BLK
_C1 = 0.7978845608028654
_C2 = _C1 * 0.044715


def _gelu2(x):
    one = jnp.asarray(1.0, x.dtype)
    c1 = jnp.asarray(_C1, x.dtype)
    c2 = jnp.asarray(_C2, x.dtype)
    t = jnp.tanh(x * (c1 + c2 * (x * x)))
    return x * (one + t)


def _mlp_half(x_ref, w1, b1, w2, b2):
    h = _gelu2(
        jnp.dot(x_ref[...].astype(jnp.bfloat16), w1,
                preferred_element_type=jnp.float32).astype(jnp.bfloat16)
        + b1)
    return jnp.dot(h, w2, preferred_element_type=jnp.float32) + b2


def _mlp_body(xq_ref, xe_ref, w1_ref, b1_ref, w2_ref, b2_ref,
              oq_ref, oe_ref):
    w1 = w1_ref[...]
    w2 = w2_ref[...]
    b1 = b1_ref[...]
    b2 = b2_ref[...]
    oq_ref[...] = _mlp_half(xq_ref, w1, b1, w2, b2)
    oe_ref[...] = _mlp_half(xe_ref, w1, b1, w2, b2)


def _mlp_alias_body(xq_ref, xe_ref, w1_ref, b1_ref, w2_ref, b2_ref,
                    _aq_ref, _ae_ref, oq_ref, oe_ref):
    _mlp_body(xq_ref, xe_ref, w1_ref, b1_ref, w2_ref, b2_ref,
              oq_ref, oe_ref)


def _mlp_chunk(chunk, body_args, prev):
    off = chunk * _GSTEPS
    row_in = pl.BlockSpec((_BLK, _D), lambda i: (i, 0))
    row_out = pl.BlockSpec((_BLK, _D), lambda i, o=off: (i + o, 0))
    in_specs = [
        row_in,
        row_in,
        pl.BlockSpec((_D, _H), lambda i: (0, 0)),
        pl.BlockSpec((1, _H), lambda i: (0, 0)),
        pl.BlockSpec((_H, _D), lambda i: (0, 0)),
        pl.BlockSpec((1, _D), lambda i: (0, 0)),
    ]
    body = _mlp_body
    aliases = {}
    args = tuple(body_args)
    if chunk > 0:
        body = _mlp_alias_body
        in_specs += [pl.BlockSpec(memory_space=pl.ANY),
                     pl.BlockSpec(memory_space=pl.ANY)]
        aliases = {6: 0, 7: 1}
        args = args + tuple(prev)
    return pl.pallas_call(
        body,
        grid=(_GSTEPS,),
        in_specs=in_specs,
        out_specs=[row_out, row_out],
        out_shape=[
            jax.ShapeDtypeStruct((_B, _D), jnp.float32),
            jax.ShapeDtypeStruct((_B, _D), jnp.float32),
        ],
        input_output_aliases=aliases,
    )(*args)


@jax.jit
def kernel(query_ids, entity_ids, query_emb, ent_emb, W1, b1, W2, b2):
    qids = query_ids.astype(jnp.int32)
    eids = entity_ids.astype(jnp.int32)
    w1 = W1.astype(jnp.bfloat16)
    w2 = (0.5 * W2).astype(jnp.bfloat16)
    b1r = b1.astype(jnp.bfloat16).reshape(1, _H)
    b2r = b2.reshape(1, _D)
    gather = _gather_call()
    rows = [
        gather(query_emb, ent_emb,
               qids[c * _CB:(c + 1) * _CB], eids[c * _CB:(c + 1) * _CB])
        for c in range(_NCHUNK)
    ]
    out = None
    for c in range(_NCHUNK):
        qr, er = rows[c]
        out = _mlp_chunk(c, (qr, er, w1, b1r, w2, b2r), out)
    return out


# speedup vs baseline: 1.0741x; 1.0741x over previous
import functools

import jax
import jax.numpy as jnp
from jax import lax
from jax.experimental import pallas as pl
from jax.experimental.pallas import tpu as pltpu
from jax.experimental.pallas import tpu_sc as plsc

_VOCAB = 100000
_D = 128
_H = 512
_B = 16384

_NCHUNK = 2
_CB = _B // _NCHUNK

_NC, _NS = 2, 16
_NW = _NC * _NS
_BPW = _CB // _NW
_CROWS = min(_BPW, 128)
_CH = _BPW // _CROWS


def _sc_gather(qtab_hbm, etab_hbm, qids_hbm, eids_hbm, q_out, e_out,
               qidx_v, eidx_v, rows0, rows1, sem0, sem1):
    wid = lax.axis_index("s") * _NC + lax.axis_index("c")
    base = wid * _BPW
    pltpu.sync_copy(qids_hbm.at[pl.ds(base, _BPW)], qidx_v)
    pltpu.sync_copy(eids_hbm.at[pl.ds(base, _BPW)], eidx_v)
    bufs = (rows0, rows1)
    sems = (sem0, sem1)
    work = [(qtab_hbm, qidx_v, q_out, c) for c in range(_CH)]
    work += [(etab_hbm, eidx_v, e_out, c) for c in range(_CH)]

    def _start(j, buf, sem):
        tab, idx, _, c = work[j]
        pltpu.async_copy(tab.at[idx.at[pl.ds(c * _CROWS, _CROWS)]], buf, sem)

    _start(0, bufs[0], sems[0])
    for j in range(len(work)):
        if j + 1 < len(work):
            _start(j + 1, bufs[(j + 1) % 2], sems[(j + 1) % 2])
        tab, idx, out, c = work[j]
        buf = bufs[j % 2]
        pltpu.make_async_copy(tab.at[idx.at[pl.ds(c * _CROWS, _CROWS)]],
                              buf, sems[j % 2]).wait()
        pltpu.sync_copy(buf, out.at[pl.ds(base + c * _CROWS, _CROWS)])


@functools.lru_cache(maxsize=None)
def _gather_call():
    return pl.kernel(
        _sc_gather,
        mesh=plsc.VectorSubcoreMesh(core_axis_name="c", subcore_axis_name="s"),
        out_type=[
            jax.ShapeDtypeStruct((_CB, _D), jnp.float32),
            jax.ShapeDtypeStruct((_CB, _D), jnp.float32),
        ],
        scratch_types=[
            pltpu.VMEM((_BPW,), jnp.int32),
            pltpu.VMEM((_BPW,), jnp.int32),
            pltpu.VMEM((_CROWS, _D), jnp.float32),
            pltpu.VMEM((_CROWS, _D), jnp.float32),
            pltpu.SemaphoreType.DMA,
            pltpu.SemaphoreType.DMA,
        ],
    )


_BLK = 2048
_GSTEPS = _CB // _BLK
_C1 = 0.7978845608028654
_C2 = _C1 * 0.044715


def _gelu2(x):
    one = jnp.asarray(1.0, x.dtype)
    c1 = jnp.asarray(_C1, x.dtype)
    c2 = jnp.asarray(_C2, x.dtype)
    t = jnp.tanh(x * (c1 + c2 * (x * x)))
    return x * (one + t)


def _mlp_half(x_ref, w1, b1, w2, b2):
    h = _gelu2(
        jnp.dot(x_ref[...].astype(jnp.bfloat16), w1,
                preferred_element_type=jnp.float32).astype(jnp.bfloat16)
        + b1)
    return jnp.dot(h, w2, preferred_element_type=jnp.float32) + b2


def _mlp_body(xq_ref, xe_ref, w1_ref, b1_ref, w2_ref, b2_ref,
              oq_ref, oe_ref):
    w1 = w1_ref[...]
    w2 = w2_ref[...]
    b1 = b1_ref[...]
    b2 = b2_ref[...]
    oq_ref[...] = _mlp_half(xq_ref, w1, b1, w2, b2)
    oe_ref[...] = _mlp_half(xe_ref, w1, b1, w2, b2)


def _mlp_alias_body(xq_ref, xe_ref, w1_ref, b1_ref, w2_ref, b2_ref,
                    _aq_ref, _ae_ref, oq_ref, oe_ref):
    _mlp_body(xq_ref, xe_ref, w1_ref, b1_ref, w2_ref, b2_ref,
              oq_ref, oe_ref)


def _mlp_chunk(chunk, body_args, prev):
    off = chunk * _GSTEPS
    row_in = pl.BlockSpec((_BLK, _D), lambda i: (i, 0))
    row_out = pl.BlockSpec((_BLK, _D), lambda i, o=off: (i + o, 0))
    in_specs = [
        row_in,
        row_in,
        pl.BlockSpec((_D, _H), lambda i: (0, 0)),
        pl.BlockSpec((1, _H), lambda i: (0, 0)),
        pl.BlockSpec((_H, _D), lambda i: (0, 0)),
        pl.BlockSpec((1, _D), lambda i: (0, 0)),
    ]
    body = _mlp_body
    aliases = {}
    args = tuple(body_args)
    if chunk > 0:
        body = _mlp_alias_body
        in_specs += [pl.BlockSpec(memory_space=pl.ANY),
                     pl.BlockSpec(memory_space=pl.ANY)]
        aliases = {6: 0, 7: 1}
        args = args + tuple(prev)
    return pl.pallas_call(
        body,
        grid=(_GSTEPS,),
        in_specs=in_specs,
        out_specs=[row_out, row_out],
        out_shape=[
            jax.ShapeDtypeStruct((_B, _D), jnp.float32),
            jax.ShapeDtypeStruct((_B, _D), jnp.float32),
        ],
        input_output_aliases=aliases,
    )(*args)


@jax.jit
def kernel(query_ids, entity_ids, query_emb, ent_emb, W1, b1, W2, b2):
    qids = query_ids.astype(jnp.int32)
    eids = entity_ids.astype(jnp.int32)
    w1 = W1.astype(jnp.bfloat16)
    w2 = (0.5 * W2).astype(jnp.bfloat16)
    b1r = b1.astype(jnp.bfloat16).reshape(1, _H)
    b2r = b2.reshape(1, _D)
    gather = _gather_call()
    rows = [
        gather(query_emb, ent_emb,
               qids[c * _CB:(c + 1) * _CB], eids[c * _CB:(c + 1) * _CB])
        for c in range(_NCHUNK)
    ]
    out = None
    for c in range(_NCHUNK):
        qr, er = rows[c]
        out = _mlp_chunk(c, (qr, er, w1, b1r, w2, b2r), out)
    return out
